# Initial kernel scaffold; baseline (speedup 1.0000x reference)
import jax
import jax.numpy as jnp
from jax.experimental import pallas as pl


def kernel(pred_cls, pred_response, pred_bboxes, tgt_box, tgt_label, tgt_ix, tgt_iy, tgt_ibox):
    raise NotImplementedError("write your pallas kernel here")


# speedup vs baseline: 37.5252x; 37.5252x over previous
import functools

import jax
import jax.numpy as jnp
from jax import lax
from jax.experimental import pallas as pl
from jax.experimental.pallas import tpu as pltpu

B, A, CLS, H, W, T = 16, 5, 20, 19, 19, 8
S = H * W
C = A * CLS


def _sigmoid(x):
    return 1.0 / (1.0 + jnp.exp(-x))


def _bce(x, t):
    return jnp.maximum(x, 0.0) - x * t + jnp.log(1.0 + jnp.exp(-jnp.abs(x)))


def _loss_kernel(cls_ref, resp_ref, bb_ref, tb_ref, lbl_ref, tix_ref, tiy_ref,
                 tib_ref, out_ref):
    cls_f = cls_ref[...]
    resp_f = resp_ref[...]
    bb_f = bb_ref[...]
    tb = tb_ref[...]
    lbl = lbl_ref[...]
    tix = tix_ref[...]
    tiy = tiy_ref[...]
    tib = tib_ref[...]

    tixf = tix.astype(jnp.float32)
    tiyf = tiy.astype(jnp.float32)
    s_pos = tiy * W + tix
    iota_S = lax.broadcasted_iota(jnp.int32, (1, 1, S), 2)

    p = tib * S + s_pos
    eq = (p[:, :, None] == p[:, None, :])
    tpos = lax.broadcasted_iota(jnp.int32, (T, T), 0)
    tpos2 = lax.broadcasted_iota(jnp.int32, (T, T), 1)
    later = (tpos2 > tpos)[None, :, :]
    dupcnt = jnp.sum((eq & later).astype(jnp.float32), axis=-1)
    valid = dupcnt < 0.5

    loss_pos = jnp.float32(0.0)
    loss_xy = jnp.float32(0.0)
    loss_wh = jnp.float32(0.0)
    loss_cls = jnp.float32(0.0)

    iota_ch_bb = lax.broadcasted_iota(jnp.int32, (1, 4, A * 4), 2)
    c_iota4 = lax.broadcasted_iota(jnp.int32, (1, 4, 1), 1)
    iota_ch_cls = lax.broadcasted_iota(jnp.int32, (1, CLS, C), 2)
    c_iota20 = lax.broadcasted_iota(jnp.int32, (1, CLS, 1), 1)
    iota_A = lax.broadcasted_iota(jnp.int32, (1, A), 1)
    iota_cls = lax.broadcasted_iota(jnp.int32, (1, CLS), 1)

    for t in range(T):
        s_t = s_pos[:, t][:, None]
        ib_t = tib[:, t][:, None]
        seq = (iota_S == s_t[:, :, None]).astype(jnp.float32)

        val_bb = jnp.sum(bb_f * seq, axis=-1)
        mask_bb = (iota_ch_bb == (ib_t[:, :, None] * 4 + c_iota4)).astype(jnp.float32)
        off = jnp.sum(val_bb[:, None, :] * mask_bb, axis=-1)

        val_cls = jnp.sum(cls_f * seq, axis=-1)
        mask_cls = (iota_ch_cls == (ib_t[:, :, None] * CLS + c_iota20)).astype(jnp.float32)
        logits = jnp.sum(val_cls[:, None, :] * mask_cls, axis=-1)

        val_resp = jnp.sum(resp_f * seq, axis=-1)
        resp_t = jnp.sum(val_resp * (iota_A == ib_t).astype(jnp.float32),
                         axis=-1, keepdims=True)

        tb_t = tb[:, t, :]
        tbx, tby = tb_t[:, 0][:, None], tb_t[:, 1][:, None]
        tbw, tbh = tb_t[:, 2][:, None], tb_t[:, 3][:, None]
        ox, oy = off[:, 0][:, None], off[:, 1][:, None]
        ow, oh = off[:, 2][:, None], off[:, 3][:, None]
        xf, yf = tixf[:, t][:, None], tiyf[:, t][:, None]

        px1 = _sigmoid(ox) + xf - ow * 0.5
        py1 = _sigmoid(oy) + yf - oh * 0.5
        gx1 = tbx + xf - tbw * 0.5
        gy1 = tby + yf - tbh * 0.5
        dx = jnp.maximum(jnp.minimum(px1 + ow, gx1 + tbw) - jnp.maximum(px1, gx1), 0.0)
        dy = jnp.maximum(jnp.minimum(py1 + oh, gy1 + tbh) - jnp.maximum(py1, gy1), 0.0)
        inter = dx * dy
        union = ow * oh + tbw * tbh - inter
        iou_t = inter / union

        vmask = valid[:, t][:, None]
        loss_pos += jnp.sum(jnp.where(vmask, _bce(resp_t, iou_t), 0.0))
        loss_xy += jnp.sum(_bce(ox, tbx) + _bce(oy, tby))
        loss_wh += jnp.sum((ow - tbw) ** 2 + (oh - tbh) ** 2)

        m = jnp.max(logits, axis=-1, keepdims=True)
        lse = m[:, 0] + jnp.log(jnp.sum(jnp.exp(logits - m), axis=-1))
        picked = jnp.sum(logits * (iota_cls == lbl[:, t][:, None]).astype(jnp.float32),
                         axis=-1)
        loss_cls += jnp.sum(lse - picked)

    bb4 = bb_f.reshape(B, A, 4, S)
    ox, oy = bb4[:, :, 0, :], bb4[:, :, 1, :]
    ow, oh = bb4[:, :, 2, :], bb4[:, :, 3, :]
    Xc = (iota_S % W).astype(jnp.float32)
    Yc = (iota_S // W).astype(jnp.float32)
    tbl = tb[:, T - 1, :]
    lx = tixf[:, T - 1][:, None, None]
    ly = tiyf[:, T - 1][:, None, None]
    Gx1 = tbl[:, 0][:, None, None] + lx - tbl[:, 2][:, None, None] * 0.5
    Gy1 = tbl[:, 1][:, None, None] + ly - tbl[:, 3][:, None, None] * 0.5
    GW = tbl[:, 2][:, None, None]
    GH = tbl[:, 3][:, None, None]
    Px1 = _sigmoid(ox) + Xc - ow * 0.5
    Py1 = _sigmoid(oy) + Yc - oh * 0.5
    DX = jnp.maximum(jnp.minimum(Px1 + ow, Gx1 + GW) - jnp.maximum(Px1, Gx1), 0.0)
    DY = jnp.maximum(jnp.minimum(Py1 + oh, Gy1 + GH) - jnp.maximum(Py1, Gy1), 0.0)
    INTER = DX * DY
    UNION = ow * oh + GW * GH - INTER
    negm = (INTER / UNION) < 0.6

    iota_A3 = lax.broadcasted_iota(jnp.int32, (1, A, 1), 1)
    posmap = jnp.zeros((B, A, S), dtype=jnp.float32)
    for t in range(T):
        aeq = (iota_A3 == tib[:, t][:, None, None])
        seq = (iota_S == s_pos[:, t][:, None, None])
        posmap += (aeq & seq).astype(jnp.float32)
    softp = jnp.maximum(resp_f, 0.0) + jnp.log(1.0 + jnp.exp(-jnp.abs(resp_f)))
    loss_neg = 0.5 * jnp.sum(jnp.where(negm & (posmap < 0.5), softp, 0.0))

    inv_b = 1.0 / B
    lane = lax.broadcasted_iota(jnp.int32, (1, 128), 1)
    out = (jnp.where(lane == 0, loss_pos * inv_b, 0.0)
           + jnp.where(lane == 1, loss_neg * inv_b, 0.0)
           + jnp.where(lane == 2, loss_cls * inv_b, 0.0)
           + jnp.where(lane == 3, loss_xy * inv_b, 0.0)
           + jnp.where(lane == 4, loss_wh * inv_b * 5.0, 0.0))
    out_ref[...] = out


@jax.jit
def kernel(pred_cls, pred_response, pred_bboxes, tgt_box, tgt_label, tgt_ix,
           tgt_iy, tgt_ibox):
    cls_f = pred_cls.reshape(B, C, S)
    resp_f = pred_response.reshape(B, A, S)
    bb_f = pred_bboxes.reshape(B, A * 4, S)

    out = pl.pallas_call(
        _loss_kernel,
        out_shape=jax.ShapeDtypeStruct((1, 128), jnp.float32),
    )(cls_f, resp_f, bb_f, tgt_box, tgt_label.astype(jnp.int32),
      tgt_ix.astype(jnp.int32), tgt_iy.astype(jnp.int32),
      tgt_ibox.astype(jnp.int32))
    return out[0, :5]


# speedup vs baseline: 60.2731x; 1.6062x over previous
import functools

import jax
import jax.numpy as jnp
from jax import lax
from jax.experimental import pallas as pl
from jax.experimental.pallas import tpu as pltpu
from jax.experimental.pallas import tpu_sc as plsc

B, A, CLS, H, W, T = 16, 5, 20, 19, 19, 8
S = H * W
C = A * CLS
NC, NS, LANES = 2, 16, 16


def _sigmoid(x):
    return 1.0 / (1.0 + jnp.exp(-x))


def _bce(x, t):
    return jnp.maximum(x, 0.0) - x * t + jnp.log(1.0 + jnp.exp(-jnp.abs(x)))


def _sc_gather_kernel(cls_hbm, bb_hbm, resp_hbm, tix_hbm, tiy_hbm, tib_hbm,
                      out_cls, out_off, out_resp,
                      tix_v, tiy_v, tib_v, lg_v, off_v, resp_v, sem):
    wid = lax.axis_index("s") * NC + lax.axis_index("c")

    @pl.when(wid < B)
    def _():
        b = wid
        base8 = pl.multiple_of(b * T, 8)
        pltpu.sync_copy(tix_hbm.at[pl.ds(base8, T)], tix_v)
        pltpu.sync_copy(tiy_hbm.at[pl.ds(base8, T)], tiy_v)
        pltpu.sync_copy(tib_hbm.at[pl.ds(base8, T)], tib_v)

        lane = lax.iota(jnp.int32, LANES)
        copies = []

        for v in range(10):
            j = v * LANES + lane
            t_c = j // CLS
            c_c = j - t_c * CLS
            ib = plsc.load_gather(tib_v, [t_c])
            ix = plsc.load_gather(tix_v, [t_c])
            iy = plsc.load_gather(tiy_v, [t_c])
            cidx = b * (C * S) + (ib * CLS + c_c) * S + iy * W + ix
            copies.append(
                pltpu.async_copy(cls_hbm.at[cidx],
                                 lg_v.at[pl.ds(v * LANES, LANES)], sem))

        for v in range(2):
            t_o = lane // 4 + v * 4
            c_o = lane % 4
            ib = plsc.load_gather(tib_v, [t_o])
            ix = plsc.load_gather(tix_v, [t_o])
            iy = plsc.load_gather(tiy_v, [t_o])
            oidx = b * (4 * A * S) + (ib * 4 + c_o) * S + iy * W + ix
            copies.append(
                pltpu.async_copy(bb_hbm.at[oidx],
                                 off_v.at[pl.ds(v * LANES, LANES)], sem))

        t_r = jnp.minimum(lane, T - 1)
        ib = plsc.load_gather(tib_v, [t_r])
        ix = plsc.load_gather(tix_v, [t_r])
        iy = plsc.load_gather(tiy_v, [t_r])
        ridx = b * (A * S) + ib * S + iy * W + ix
        copies.append(pltpu.async_copy(resp_hbm.at[ridx], resp_v, sem))

        for cp in copies:
            cp.wait()

        pltpu.sync_copy(lg_v, out_cls.at[pl.ds(pl.multiple_of(b * 160, 8), 160)])
        pltpu.sync_copy(off_v, out_off.at[pl.ds(pl.multiple_of(b * 32, 8), 32)])
        pltpu.sync_copy(resp_v.at[pl.ds(0, T)],
                        out_resp.at[pl.ds(base8, T)])


def _sc_gather(cls_flat, bb_flat, resp_flat, tix, tiy, tib):
    mesh = plsc.VectorSubcoreMesh(core_axis_name="c", subcore_axis_name="s")
    fn = functools.partial(
        pl.kernel,
        out_type=(
            jax.ShapeDtypeStruct((B * T * CLS,), jnp.float32),
            jax.ShapeDtypeStruct((B * T * 4,), jnp.float32),
            jax.ShapeDtypeStruct((B * T,), jnp.float32),
        ),
        mesh=mesh,
        scratch_types=[
            pltpu.VMEM((T,), jnp.int32),
            pltpu.VMEM((T,), jnp.int32),
            pltpu.VMEM((T,), jnp.int32),
            pltpu.VMEM((T * CLS,), jnp.float32),
            pltpu.VMEM((T * 4,), jnp.float32),
            pltpu.VMEM((LANES,), jnp.float32),
            pltpu.SemaphoreType.DMA,
        ],
        compiler_params=pltpu.CompilerParams(needs_layout_passes=False),
    )(_sc_gather_kernel)
    return fn(cls_flat, bb_flat, resp_flat, tix, tiy, tib)


def _loss_kernel(resp_ref, bb_ref, tb_ref, lbl_ref, tix_ref, tiy_ref, tib_ref,
                 glog_ref, goff_ref, gresp_ref, out_ref):
    resp_f = resp_ref[...]
    bb_f = bb_ref[...]
    tb = tb_ref[...]
    lbl = lbl_ref[...]
    tix = tix_ref[...]
    tiy = tiy_ref[...]
    tib = tib_ref[...]
    glog = glog_ref[...]
    goff = goff_ref[...]
    resp_t = gresp_ref[...]

    tixf = tix.astype(jnp.float32)
    tiyf = tiy.astype(jnp.float32)
    s_pos = tiy * W + tix
    iota_S = lax.broadcasted_iota(jnp.int32, (1, 1, S), 2)

    p = tib * S + s_pos
    eq = (p[:, :, None] == p[:, None, :])
    tpos = lax.broadcasted_iota(jnp.int32, (T, T), 0)
    tpos2 = lax.broadcasted_iota(jnp.int32, (T, T), 1)
    later = (tpos2 > tpos)[None, :, :]
    dupcnt = jnp.sum((eq & later).astype(jnp.float32), axis=-1)
    valid = dupcnt < 0.5

    ox, oy = goff[:, :, 0], goff[:, :, 1]
    ow, oh = goff[:, :, 2], goff[:, :, 3]
    tbx, tby = tb[:, :, 0], tb[:, :, 1]
    tbw, tbh = tb[:, :, 2], tb[:, :, 3]
    px1 = _sigmoid(ox) + tixf - ow * 0.5
    py1 = _sigmoid(oy) + tiyf - oh * 0.5
    gx1 = tbx + tixf - tbw * 0.5
    gy1 = tby + tiyf - tbh * 0.5
    dx = jnp.maximum(jnp.minimum(px1 + ow, gx1 + tbw) - jnp.maximum(px1, gx1), 0.0)
    dy = jnp.maximum(jnp.minimum(py1 + oh, gy1 + tbh) - jnp.maximum(py1, gy1), 0.0)
    inter = dx * dy
    union = ow * oh + tbw * tbh - inter
    iou_t = inter / union

    loss_pos = jnp.sum(jnp.where(valid, _bce(resp_t, iou_t), 0.0))
    loss_xy = jnp.sum(_bce(ox, tbx) + _bce(oy, tby))
    loss_wh = jnp.sum((ow - tbw) ** 2 + (oh - tbh) ** 2)

    m = jnp.max(glog, axis=-1, keepdims=True)
    lse = m[:, :, 0] + jnp.log(jnp.sum(jnp.exp(glog - m), axis=-1))
    iota_cls = lax.broadcasted_iota(jnp.int32, (1, 1, CLS), 2)
    picked = jnp.sum(glog * (iota_cls == lbl[:, :, None]).astype(jnp.float32),
                     axis=-1)
    loss_cls = jnp.sum(lse - picked)

    bb4 = bb_f.reshape(B, A, 4, S)
    mox, moy = bb4[:, :, 0, :], bb4[:, :, 1, :]
    mow, moh = bb4[:, :, 2, :], bb4[:, :, 3, :]
    Xc = (iota_S % W).astype(jnp.float32)
    Yc = (iota_S // W).astype(jnp.float32)
    tbl = tb[:, T - 1, :]
    lx = tixf[:, T - 1][:, None, None]
    ly = tiyf[:, T - 1][:, None, None]
    Gx1 = tbl[:, 0][:, None, None] + lx - tbl[:, 2][:, None, None] * 0.5
    Gy1 = tbl[:, 1][:, None, None] + ly - tbl[:, 3][:, None, None] * 0.5
    GW = tbl[:, 2][:, None, None]
    GH = tbl[:, 3][:, None, None]
    Px1 = _sigmoid(mox) + Xc - mow * 0.5
    Py1 = _sigmoid(moy) + Yc - moh * 0.5
    DX = jnp.maximum(jnp.minimum(Px1 + mow, Gx1 + GW) - jnp.maximum(Px1, Gx1), 0.0)
    DY = jnp.maximum(jnp.minimum(Py1 + moh, Gy1 + GH) - jnp.maximum(Py1, Gy1), 0.0)
    INTER = DX * DY
    UNION = mow * moh + GW * GH - INTER
    negm = (INTER / UNION) < 0.6

    iota_A3 = lax.broadcasted_iota(jnp.int32, (1, A, 1), 1)
    posmap = jnp.zeros((B, A, S), dtype=jnp.float32)
    for t in range(T):
        aeq = (iota_A3 == tib[:, t][:, None, None])
        seq = (iota_S == s_pos[:, t][:, None, None])
        posmap += (aeq & seq).astype(jnp.float32)
    softp = jnp.maximum(resp_f, 0.0) + jnp.log(1.0 + jnp.exp(-jnp.abs(resp_f)))
    loss_neg = 0.5 * jnp.sum(jnp.where(negm & (posmap < 0.5), softp, 0.0))

    inv_b = 1.0 / B
    lane = lax.broadcasted_iota(jnp.int32, (1, 128), 1)
    out = (jnp.where(lane == 0, loss_pos * inv_b, 0.0)
           + jnp.where(lane == 1, loss_neg * inv_b, 0.0)
           + jnp.where(lane == 2, loss_cls * inv_b, 0.0)
           + jnp.where(lane == 3, loss_xy * inv_b, 0.0)
           + jnp.where(lane == 4, loss_wh * inv_b * 5.0, 0.0))
    out_ref[...] = out


@jax.jit
def kernel(pred_cls, pred_response, pred_bboxes, tgt_box, tgt_label, tgt_ix,
           tgt_iy, tgt_ibox):
    resp_f = pred_response.reshape(B, A, S)
    bb_f = pred_bboxes.reshape(B, A * 4, S)
    tix = tgt_ix.astype(jnp.int32)
    tiy = tgt_iy.astype(jnp.int32)
    tib = tgt_ibox.astype(jnp.int32)

    g_cls, g_off, g_resp = _sc_gather(
        pred_cls.reshape(B * C * S), pred_bboxes.reshape(B * 4 * A * S),
        pred_response.reshape(B * A * S),
        tix.reshape(B * T), tiy.reshape(B * T), tib.reshape(B * T))

    out = pl.pallas_call(
        _loss_kernel,
        out_shape=jax.ShapeDtypeStruct((1, 128), jnp.float32),
    )(resp_f, bb_f, tgt_box, tgt_label.astype(jnp.int32), tix, tiy, tib,
      g_cls.reshape(B, T, CLS), g_off.reshape(B, T, 4), g_resp.reshape(B, T))
    return out[0, :5]


# speedup vs baseline: 74.4943x; 1.2359x over previous
import functools

import jax
import jax.numpy as jnp
from jax import lax
from jax.experimental import pallas as pl
from jax.experimental.pallas import tpu as pltpu
from jax.experimental.pallas import tpu_sc as plsc

B, A, CLS, H, W, T = 16, 5, 20, 19, 19, 8
S = H * W
C = A * CLS
NC, NS, LANES = 2, 16, 16


def _sigmoid(x):
    return 1.0 / (1.0 + jnp.exp(-x))


def _bce(x, t):
    return jnp.maximum(x, 0.0) - x * t + jnp.log(1.0 + jnp.exp(-jnp.abs(x)))


def _sc_gather_kernel(bb_hbm, resp_hbm, tix_hbm, tiy_hbm, tib_hbm,
                      out_off, out_resp,
                      tix_v, tiy_v, tib_v, off_v, resp_v, sem):
    wid = lax.axis_index("s") * NC + lax.axis_index("c")

    @pl.when(wid < B)
    def _():
        b = wid
        base8 = pl.multiple_of(b * T, 8)
        pltpu.sync_copy(tix_hbm.at[pl.ds(base8, T)], tix_v)
        pltpu.sync_copy(tiy_hbm.at[pl.ds(base8, T)], tiy_v)
        pltpu.sync_copy(tib_hbm.at[pl.ds(base8, T)], tib_v)

        lane = lax.iota(jnp.int32, LANES)
        copies = []

        for v in range(2):
            t_o = lane // 4 + v * 4
            c_o = lane % 4
            ib = plsc.load_gather(tib_v, [t_o])
            ix = plsc.load_gather(tix_v, [t_o])
            iy = plsc.load_gather(tiy_v, [t_o])
            oidx = b * (4 * A * S) + (ib * 4 + c_o) * S + iy * W + ix
            copies.append(
                pltpu.async_copy(bb_hbm.at[oidx],
                                 off_v.at[pl.ds(v * LANES, LANES)], sem))

        t_r = jnp.minimum(lane, T - 1)
        ib = plsc.load_gather(tib_v, [t_r])
        ix = plsc.load_gather(tix_v, [t_r])
        iy = plsc.load_gather(tiy_v, [t_r])
        ridx = b * (A * S) + ib * S + iy * W + ix
        copies.append(pltpu.async_copy(resp_hbm.at[ridx], resp_v, sem))

        for cp in copies:
            cp.wait()

        pltpu.sync_copy(off_v, out_off.at[pl.ds(pl.multiple_of(b * 32, 8), 32)])
        pltpu.sync_copy(resp_v.at[pl.ds(0, T)], out_resp.at[pl.ds(base8, T)])


def _sc_gather(bb_flat, resp_flat, tix, tiy, tib):
    mesh = plsc.VectorSubcoreMesh(core_axis_name="c", subcore_axis_name="s")
    fn = functools.partial(
        pl.kernel,
        out_type=(
            jax.ShapeDtypeStruct((B * T * 4,), jnp.float32),
            jax.ShapeDtypeStruct((B * T,), jnp.float32),
        ),
        mesh=mesh,
        scratch_types=[
            pltpu.VMEM((T,), jnp.int32),
            pltpu.VMEM((T,), jnp.int32),
            pltpu.VMEM((T,), jnp.int32),
            pltpu.VMEM((T * 4,), jnp.float32),
            pltpu.VMEM((LANES,), jnp.float32),
            pltpu.SemaphoreType.DMA,
        ],
        compiler_params=pltpu.CompilerParams(needs_layout_passes=False),
    )(_sc_gather_kernel)
    return fn(bb_flat, resp_flat, tix, tiy, tib)


def _loss_kernel(cls_ref, resp_ref, bb_ref, tb_ref, lbl128_ref, tix_ref,
                 tiy_ref, tib_ref, tix_s, tiy_s, tib_s,
                 goff_ref, gresp_ref, out_ref, lg_scratch):
    tb = tb_ref[...]
    tix = tix_ref[...]
    tiy = tiy_ref[...]
    tib = tib_ref[...]
    goff = goff_ref[...]
    resp_t = gresp_ref[...]

    tixf = tix.astype(jnp.float32)
    tiyf = tiy.astype(jnp.float32)

    p = tib * S + tiy * W + tix
    eq = (p[:, :, None] == p[:, None, :])
    tpos = lax.broadcasted_iota(jnp.int32, (T, T), 0)
    tpos2 = lax.broadcasted_iota(jnp.int32, (T, T), 1)
    dup_later = jnp.sum((eq & (tpos2 > tpos)[None]).astype(jnp.float32), -1)
    valid_last = dup_later < 0.5
    dup_earlier = jnp.sum((eq & (tpos2 < tpos)[None]).astype(jnp.float32), -1)
    valid_first = dup_earlier < 0.5

    ox, oy = goff[:, :, 0], goff[:, :, 1]
    ow, oh = goff[:, :, 2], goff[:, :, 3]
    tbx, tby = tb[:, :, 0], tb[:, :, 1]
    tbw, tbh = tb[:, :, 2], tb[:, :, 3]
    px1 = _sigmoid(ox) + tixf - ow * 0.5
    py1 = _sigmoid(oy) + tiyf - oh * 0.5
    gx1 = tbx + tixf - tbw * 0.5
    gy1 = tby + tiyf - tbh * 0.5
    dx = jnp.maximum(jnp.minimum(px1 + ow, gx1 + tbw) - jnp.maximum(px1, gx1), 0.0)
    dy = jnp.maximum(jnp.minimum(py1 + oh, gy1 + tbh) - jnp.maximum(py1, gy1), 0.0)
    inter = dx * dy
    iou_t = inter / (ow * oh + tbw * tbh - inter)

    loss_pos = jnp.sum(jnp.where(valid_last, _bce(resp_t, iou_t), 0.0))
    loss_xy = jnp.sum(_bce(ox, tbx) + _bce(oy, tby))
    loss_wh = jnp.sum((ow - tbw) ** 2 + (oh - tbh) ** 2)

    ltbx = tb[:, T - 1, 0][:, None]
    ltby = tb[:, T - 1, 1][:, None]
    ltbw = tb[:, T - 1, 2][:, None]
    ltbh = tb[:, T - 1, 3][:, None]
    lgx1 = ltbx + tixf[:, T - 1][:, None] - ltbw * 0.5
    lgy1 = ltby + tiyf[:, T - 1][:, None] - ltbh * 0.5
    ldx = jnp.maximum(jnp.minimum(px1 + ow, lgx1 + ltbw) - jnp.maximum(px1, lgx1), 0.0)
    ldy = jnp.maximum(jnp.minimum(py1 + oh, lgy1 + ltbh) - jnp.maximum(py1, lgy1), 0.0)
    linter = ldx * ldy
    iou_last_t = linter / (ow * oh + ltbw * ltbh - linter)

    softp_t = jnp.maximum(resp_t, 0.0) + jnp.log(1.0 + jnp.exp(-jnp.abs(resp_t)))
    sub_neg = jnp.sum(jnp.where(valid_first & (iou_last_t < 0.6), softp_t, 0.0))

    xiota = lax.broadcasted_iota(jnp.int32, (1, W), 1)
    for b in range(B):
        for t in range(T):
            ib = tib_s[b, t]
            iy = tiy_s[b, t]
            ix = tix_s[b, t]
            blk = cls_ref[b, pl.ds(ib * CLS, CLS), pl.ds(iy, 1), :]
            vals = jnp.sum(blk[:, 0, :] * (xiota == ix).astype(jnp.float32),
                           axis=1, keepdims=True)
            col = b * T + t
            lg_scratch[:, col:col + 1] = vals

    glog = lg_scratch[...]
    mx = jnp.max(glog, axis=0, keepdims=True)
    lse = mx + jnp.log(jnp.sum(jnp.exp(glog - mx), axis=0, keepdims=True))
    sub_iota = lax.broadcasted_iota(jnp.int32, (CLS, B * T), 0)
    picked = jnp.sum(glog * (sub_iota == lbl128_ref[...]).astype(jnp.float32),
                     axis=0, keepdims=True)
    loss_cls = jnp.sum(lse - picked)

    resp_f = resp_ref[...]
    bb4 = bb_ref[...].reshape(B, A, 4, H, W)
    mox, moy = bb4[:, :, 0], bb4[:, :, 1]
    mow, moh = bb4[:, :, 2], bb4[:, :, 3]
    Xc = lax.broadcasted_iota(jnp.int32, (1, 1, 1, W), 3).astype(jnp.float32)
    Yc = lax.broadcasted_iota(jnp.int32, (1, 1, H, 1), 2).astype(jnp.float32)
    Gx1 = (ltbx + tixf[:, T - 1][:, None] - ltbw * 0.5)[:, :, None, None]
    Gy1 = (ltby + tiyf[:, T - 1][:, None] - ltbh * 0.5)[:, :, None, None]
    GW = ltbw[:, :, None, None]
    GH = ltbh[:, :, None, None]
    Px1 = _sigmoid(mox) + Xc - mow * 0.5
    Py1 = _sigmoid(moy) + Yc - moh * 0.5
    DX = jnp.maximum(jnp.minimum(Px1 + mow, Gx1 + GW) - jnp.maximum(Px1, Gx1), 0.0)
    DY = jnp.maximum(jnp.minimum(Py1 + moh, Gy1 + GH) - jnp.maximum(Py1, Gy1), 0.0)
    INTER = DX * DY
    negm = INTER / (mow * moh + GW * GH - INTER) < 0.6
    softp = jnp.maximum(resp_f, 0.0) + jnp.log(1.0 + jnp.exp(-jnp.abs(resp_f)))
    loss_neg = 0.5 * (jnp.sum(jnp.where(negm, softp, 0.0)) - sub_neg)

    inv_b = 1.0 / B
    lanev = lax.broadcasted_iota(jnp.int32, (1, 128), 1)
    out = (jnp.where(lanev == 0, loss_pos * inv_b, 0.0)
           + jnp.where(lanev == 1, loss_neg * inv_b, 0.0)
           + jnp.where(lanev == 2, loss_cls * inv_b, 0.0)
           + jnp.where(lanev == 3, loss_xy * inv_b, 0.0)
           + jnp.where(lanev == 4, loss_wh * inv_b * 5.0, 0.0))
    out_ref[...] = out


@jax.jit
def kernel(pred_cls, pred_response, pred_bboxes, tgt_box, tgt_label, tgt_ix,
           tgt_iy, tgt_ibox):
    tix = tgt_ix.astype(jnp.int32)
    tiy = tgt_iy.astype(jnp.int32)
    tib = tgt_ibox.astype(jnp.int32)
    lbl = tgt_label.astype(jnp.int32)

    g_off, g_resp = _sc_gather(
        pred_bboxes.reshape(B * 4 * A * S), pred_response.reshape(B * A * S),
        tix.reshape(B * T), tiy.reshape(B * T), tib.reshape(B * T))

    smem = pl.BlockSpec(memory_space=pltpu.SMEM)
    out = pl.pallas_call(
        _loss_kernel,
        out_shape=jax.ShapeDtypeStruct((1, 128), jnp.float32),
        in_specs=[pl.BlockSpec((B, C, H, W), lambda: (0, 0, 0, 0)),
                  pl.BlockSpec((B, A, H, W), lambda: (0, 0, 0, 0)),
                  pl.BlockSpec((B, 4 * A, H, W), lambda: (0, 0, 0, 0)),
                  pl.BlockSpec((B, T, 4), lambda: (0, 0, 0)),
                  pl.BlockSpec((1, B * T), lambda: (0, 0)),
                  pl.BlockSpec((B, T), lambda: (0, 0)),
                  pl.BlockSpec((B, T), lambda: (0, 0)),
                  pl.BlockSpec((B, T), lambda: (0, 0)),
                  smem, smem, smem,
                  pl.BlockSpec((B, T, 4), lambda: (0, 0, 0)),
                  pl.BlockSpec((B, T), lambda: (0, 0))],
        scratch_shapes=[pltpu.VMEM((CLS, B * T), jnp.float32)],
    )(pred_cls, pred_response, pred_bboxes, tgt_box, lbl.reshape(1, B * T),
      tix, tiy, tib,
      tix, tiy, tib,
      g_off.reshape(B, T, 4), g_resp.reshape(B, T))
    return out[0, :5]


# speedup vs baseline: 76.9530x; 1.0330x over previous
import functools

import jax
import jax.numpy as jnp
from jax import lax
from jax.experimental import pallas as pl
from jax.experimental.pallas import tpu as pltpu
from jax.experimental.pallas import tpu_sc as plsc

B, A, CLS, H, W, T = 16, 5, 20, 19, 19, 8
S = H * W
C = A * CLS
BT = B * T
NC, NS, LANES = 2, 16, 16


def _sigmoid(x):
    return 1.0 / (1.0 + jnp.exp(-x))


def _bce(x, t):
    return jnp.maximum(x, 0.0) - x * t + jnp.log(1.0 + jnp.exp(-jnp.abs(x)))


def _softplus(x):
    return jnp.maximum(x, 0.0) + jnp.log(1.0 + jnp.exp(-jnp.abs(x)))


def _sc_gather_kernel(resp_hbm, idx_hbm, out_resp, tix_v, tiy_v, tib_v,
                      resp_v, sem):
    wid = lax.axis_index("s") * NC + lax.axis_index("c")

    @pl.when(wid < B)
    def _():
        b = wid
        base8 = pl.multiple_of(b * T, 8)
        pltpu.sync_copy(idx_hbm.at[pl.ds(base8, T)], tix_v)
        pltpu.sync_copy(idx_hbm.at[pl.ds(base8 + BT, T)], tiy_v)
        pltpu.sync_copy(idx_hbm.at[pl.ds(base8 + 2 * BT, T)], tib_v)

        lane = lax.iota(jnp.int32, LANES)
        t_r = jnp.minimum(lane, T - 1)
        ib = plsc.load_gather(tib_v, [t_r])
        ix = plsc.load_gather(tix_v, [t_r])
        iy = plsc.load_gather(tiy_v, [t_r])
        ridx = b * (A * S) + ib * S + iy * W + ix
        pltpu.async_copy(resp_hbm.at[ridx], resp_v, sem).wait()
        pltpu.sync_copy(resp_v.at[pl.ds(0, T)], out_resp.at[pl.ds(base8, T)])


def _sc_gather(resp_flat, idx_cat):
    mesh = plsc.VectorSubcoreMesh(core_axis_name="c", subcore_axis_name="s")
    fn = functools.partial(
        pl.kernel,
        out_type=jax.ShapeDtypeStruct((BT,), jnp.float32),
        mesh=mesh,
        scratch_types=[
            pltpu.VMEM((T,), jnp.int32),
            pltpu.VMEM((T,), jnp.int32),
            pltpu.VMEM((T,), jnp.int32),
            pltpu.VMEM((LANES,), jnp.float32),
            pltpu.SemaphoreType.DMA,
        ],
        compiler_params=pltpu.CompilerParams(needs_layout_passes=False),
    )(_sc_gather_kernel)
    return fn(resp_flat, idx_cat)


def _group_bcast_last(x, tlane):
    xm = x * (tlane == T - 1).astype(jnp.float32)
    out = xm
    for t in range(T - 1):
        out = out + jnp.roll(xm, t - (T - 1), axis=1) * (tlane == t).astype(jnp.float32)
    return out


def _main_kernel(cls_ref, resp_ref, bb_ref, tbl_ref, ixyl_ref, tb4_ref,
                 lbl_ref, tixc_ref, tix_ref, tiy_ref, tib_ref, idx_s, out_ref,
                 cls_scr, off_scr, sem):
    copies = []
    for b in range(B):
        for t in range(T):
            bt = b * T + t
            iy = idx_s[BT + bt]
            ib = idx_s[2 * BT + bt]
            copies.append(pltpu.async_copy(
                cls_ref.at[b, pl.ds(ib * CLS, CLS), iy],
                cls_scr.at[bt], sem))

    xiota = lax.broadcasted_iota(jnp.int32, (1, W), 1)
    for b in range(B):
        for t in range(T):
            bt = b * T + t
            ix = idx_s[bt]
            iy = idx_s[BT + bt]
            ib = idx_s[2 * BT + bt]
            blk = bb_ref[b, pl.ds(ib * 4, 4), pl.ds(iy, 1), :]
            vals = jnp.sum(blk[:, 0, :] * (xiota == ix).astype(jnp.float32),
                           axis=1, keepdims=True)
            off_scr[:, bt:bt + 1] = vals

    lanev = lax.broadcasted_iota(jnp.int32, (1, BT), 1)
    tlane = lanev % T
    tixf = tix_ref[...].astype(jnp.float32)
    tiyf = tiy_ref[...].astype(jnp.float32)
    tib = tib_ref[...]
    p = tib * S + tiy_ref[...] * W + tix_ref[...]

    dup_later = jnp.zeros((1, BT), jnp.float32)
    dup_earlier = jnp.zeros((1, BT), jnp.float32)
    for d in range(1, T):
        eq_back = (p == jnp.roll(p, d, axis=1)) & (tlane >= d)
        dup_earlier += eq_back.astype(jnp.float32)
        eq_fwd = (p == jnp.roll(p, -d, axis=1)) & (tlane <= T - 1 - d)
        dup_later += eq_fwd.astype(jnp.float32)
    valid_last = (dup_later < 0.5).astype(jnp.float32)
    valid_first = (dup_earlier < 0.5).astype(jnp.float32)

    off = off_scr[...]
    ox, oy = off[0:1, :], off[1:2, :]
    ow, oh = off[2:3, :], off[3:4, :]
    tb4 = tb4_ref[...]
    tbx, tby, tbw, tbh = tb4[0:1], tb4[1:2], tb4[2:3], tb4[3:4]

    px1 = _sigmoid(ox) + tixf - ow * 0.5
    py1 = _sigmoid(oy) + tiyf - oh * 0.5
    gx1 = tbx + tixf - tbw * 0.5
    gy1 = tby + tiyf - tbh * 0.5
    dx = jnp.maximum(jnp.minimum(px1 + ow, gx1 + tbw) - jnp.maximum(px1, gx1), 0.0)
    dy = jnp.maximum(jnp.minimum(py1 + oh, gy1 + tbh) - jnp.maximum(py1, gy1), 0.0)
    inter = dx * dy
    iou_t = inter / (ow * oh + tbw * tbh - inter)

    loss_xy = jnp.sum(_bce(ox, tbx) + _bce(oy, tby))
    loss_wh = jnp.sum((ow - tbw) ** 2 + (oh - tbh) ** 2)

    lgx1 = _group_bcast_last(gx1, tlane)
    lgy1 = _group_bcast_last(gy1, tlane)
    ltbw = _group_bcast_last(tbw, tlane)
    ltbh = _group_bcast_last(tbh, tlane)
    ldx = jnp.maximum(jnp.minimum(px1 + ow, lgx1 + ltbw) - jnp.maximum(px1, lgx1), 0.0)
    ldy = jnp.maximum(jnp.minimum(py1 + oh, lgy1 + ltbh) - jnp.maximum(py1, lgy1), 0.0)
    linter = ldx * ldy
    iou_last_t = linter / (ow * oh + ltbw * ltbh - linter)

    resp_f = resp_ref[...]
    bb4m = bb_ref[...].reshape(B, A, 4, H, W)
    mox, moy = bb4m[:, :, 0], bb4m[:, :, 1]
    mow, moh = bb4m[:, :, 2], bb4m[:, :, 3]
    Xc = lax.broadcasted_iota(jnp.int32, (1, 1, 1, W), 3).astype(jnp.float32)
    Yc = lax.broadcasted_iota(jnp.int32, (1, 1, H, 1), 2).astype(jnp.float32)
    tbl = tbl_ref[...]
    ixyl = ixyl_ref[...].astype(jnp.float32)
    tlx = ixyl[:, 0:1]
    tly = ixyl[:, 1:2]
    Gx1 = (tbl[:, 0:1] + tlx - tbl[:, 2:3] * 0.5)[:, :, None, None]
    Gy1 = (tbl[:, 1:2] + tly - tbl[:, 3:4] * 0.5)[:, :, None, None]
    GW = tbl[:, 2:3][:, :, None, None]
    GH = tbl[:, 3:4][:, :, None, None]
    Px1 = _sigmoid(mox) + Xc - mow * 0.5
    Py1 = _sigmoid(moy) + Yc - moh * 0.5
    DX = jnp.maximum(jnp.minimum(Px1 + mow, Gx1 + GW) - jnp.maximum(Px1, Gx1), 0.0)
    DY = jnp.maximum(jnp.minimum(Py1 + moh, Gy1 + GH) - jnp.maximum(Py1, Gy1), 0.0)
    INTER = DX * DY
    negm = INTER / (mow * moh + GW * GH - INTER) < 0.6
    neg_raw = jnp.sum(jnp.where(negm, _softplus(resp_f), 0.0))

    for cp in copies:
        cp.wait()
    rows3 = cls_scr[...]
    xmask3 = (lax.broadcasted_iota(jnp.int32, (1, 1, W), 2)
              == tixc_ref[...][:, :, None]).astype(jnp.float32)
    glog = jnp.sum(rows3 * xmask3, axis=2)
    mx = jnp.max(glog, axis=1, keepdims=True)
    lse = mx + jnp.log(jnp.sum(jnp.exp(glog - mx), axis=1, keepdims=True))
    cls_iota = lax.broadcasted_iota(jnp.int32, (BT, CLS), 1)
    picked = jnp.sum(glog * (cls_iota == lbl_ref[...]).astype(jnp.float32),
                     axis=1, keepdims=True)
    loss_cls = jnp.sum(lse - picked)

    riota = lax.broadcasted_iota(jnp.int32, (8, BT), 0)
    scal = (jnp.where(lanev == 0, loss_xy, 0.0)
            + jnp.where(lanev == 1, loss_wh, 0.0)
            + jnp.where(lanev == 2, loss_cls, 0.0)
            + jnp.where(lanev == 3, neg_raw, 0.0))
    out = (jnp.where(riota == 0, iou_t, 0.0)
           + jnp.where(riota == 1, iou_last_t, 0.0)
           + jnp.where(riota == 2, valid_last, 0.0)
           + jnp.where(riota == 3, valid_first, 0.0)
           + jnp.where(riota == 4, scal, 0.0))
    out_ref[...] = out


def _epilogue_kernel(out1_ref, gresp_ref, out_ref):
    rows = out1_ref[...]
    resp = gresp_ref[...].reshape(1, BT)
    iou_t = rows[0:1, :]
    iou_last_t = rows[1:2, :]
    valid_last = rows[2:3, :]
    valid_first = rows[3:4, :]
    scal = rows[4:5, :]

    loss_pos = jnp.sum(valid_last * _bce(resp, iou_t))
    sub_neg = jnp.sum(valid_first * jnp.where(iou_last_t < 0.6,
                                              _softplus(resp), 0.0))
    lanev = lax.broadcasted_iota(jnp.int32, (1, BT), 1)

    def pick(k):
        return jnp.sum(jnp.where(lanev == k, scal, 0.0))

    loss_xy, loss_wh, loss_cls, neg_raw = pick(0), pick(1), pick(2), pick(3)
    loss_neg = 0.5 * (neg_raw - sub_neg)

    inv_b = 1.0 / B
    i5 = lax.broadcasted_iota(jnp.int32, (5,), 0)
    out = (jnp.where(i5 == 0, loss_pos * inv_b, 0.0)
           + jnp.where(i5 == 1, loss_neg * inv_b, 0.0)
           + jnp.where(i5 == 2, loss_cls * inv_b, 0.0)
           + jnp.where(i5 == 3, loss_xy * inv_b, 0.0)
           + jnp.where(i5 == 4, loss_wh * inv_b * 5.0, 0.0))
    out_ref[...] = out


@jax.jit
def kernel(pred_cls, pred_response, pred_bboxes, tgt_box, tgt_label, tgt_ix,
           tgt_iy, tgt_ibox):
    tix = tgt_ix.astype(jnp.int32)
    tiy = tgt_iy.astype(jnp.int32)
    tib = tgt_ibox.astype(jnp.int32)
    lbl = tgt_label.astype(jnp.int32)

    idx_cat = jnp.concatenate(
        [tix.reshape(BT), tiy.reshape(BT), tib.reshape(BT)])
    g_resp = _sc_gather(pred_response.reshape(B * A * S), idx_cat)

    smem = pl.BlockSpec(memory_space=pltpu.SMEM)
    out1 = pl.pallas_call(
        _main_kernel,
        out_shape=jax.ShapeDtypeStruct((8, BT), jnp.float32),
        in_specs=[pl.BlockSpec(memory_space=pltpu.HBM),
                  pl.BlockSpec((B, A, H, W), lambda: (0, 0, 0, 0)),
                  pl.BlockSpec((B, 4 * A, H, W), lambda: (0, 0, 0, 0)),
                  pl.BlockSpec((B, 4), lambda: (0, 0)),
                  pl.BlockSpec((B, 2), lambda: (0, 0)),
                  pl.BlockSpec((4, BT), lambda: (0, 0)),
                  pl.BlockSpec((BT, 1), lambda: (0, 0)),
                  pl.BlockSpec((BT, 1), lambda: (0, 0)),
                  pl.BlockSpec((1, BT), lambda: (0, 0)),
                  pl.BlockSpec((1, BT), lambda: (0, 0)),
                  pl.BlockSpec((1, BT), lambda: (0, 0)),
                  smem],
        scratch_shapes=[pltpu.VMEM((BT, CLS, W), jnp.float32),
                        pltpu.VMEM((4, BT), jnp.float32),
                        pltpu.SemaphoreType.DMA],
    )(pred_cls, pred_response, pred_bboxes,
      tgt_box[:, T - 1, :], jnp.stack([tix[:, T - 1], tiy[:, T - 1]], axis=1),
      tgt_box.reshape(BT, 4).T, lbl.reshape(BT, 1), tix.reshape(BT, 1),
      tix.reshape(1, BT), tiy.reshape(1, BT), tib.reshape(1, BT), idx_cat)

    out = pl.pallas_call(
        _epilogue_kernel,
        out_shape=jax.ShapeDtypeStruct((5,), jnp.float32),
    )(out1, g_resp)
    return out


# speedup vs baseline: 107.4493x; 1.3963x over previous
import functools

import jax
import jax.numpy as jnp
from jax import lax
from jax.experimental import pallas as pl
from jax.experimental.pallas import tpu as pltpu
from jax.experimental.pallas import tpu_sc as plsc

B, A, CLS, H, W, T = 16, 5, 20, 19, 19, 8
S = H * W
C = A * CLS
BT = B * T
NC, NS, LANES = 2, 16, 16


def _sigmoid(x):
    return 1.0 / (1.0 + jnp.exp(-x))


def _bce(x, t):
    return jnp.maximum(x, 0.0) - x * t + jnp.log(1.0 + jnp.exp(-jnp.abs(x)))


def _softplus(x):
    return jnp.maximum(x, 0.0) + jnp.log(1.0 + jnp.exp(-jnp.abs(x)))


def _sc_gather_kernel(resp_hbm, idx_hbm, out_resp, tix_v, tiy_v, tib_v,
                      resp_v, sem):
    wid = lax.axis_index("s") * NC + lax.axis_index("c")

    @pl.when(wid < B)
    def _():
        b = wid
        base8 = pl.multiple_of(b * T, 8)
        pltpu.sync_copy(idx_hbm.at[pl.ds(base8, T)], tix_v)
        pltpu.sync_copy(idx_hbm.at[pl.ds(base8 + BT, T)], tiy_v)
        pltpu.sync_copy(idx_hbm.at[pl.ds(base8 + 2 * BT, T)], tib_v)

        lane = lax.iota(jnp.int32, LANES)
        t_r = jnp.minimum(lane, T - 1)
        ib = plsc.load_gather(tib_v, [t_r])
        ix = plsc.load_gather(tix_v, [t_r])
        iy = plsc.load_gather(tiy_v, [t_r])
        ridx = b * (A * S) + ib * S + iy * W + ix
        pltpu.async_copy(resp_hbm.at[ridx], resp_v, sem).wait()
        pltpu.sync_copy(resp_v.at[pl.ds(0, T)], out_resp.at[pl.ds(base8, T)])


def _sc_gather(resp_flat, idx_cat):
    mesh = plsc.VectorSubcoreMesh(core_axis_name="c", subcore_axis_name="s")
    fn = functools.partial(
        pl.kernel,
        out_type=jax.ShapeDtypeStruct((BT,), jnp.float32),
        mesh=mesh,
        scratch_types=[
            pltpu.VMEM((T,), jnp.int32),
            pltpu.VMEM((T,), jnp.int32),
            pltpu.VMEM((T,), jnp.int32),
            pltpu.VMEM((LANES,), jnp.float32),
            pltpu.SemaphoreType.DMA,
        ],
        compiler_params=pltpu.CompilerParams(needs_layout_passes=False),
    )(_sc_gather_kernel)
    return fn(resp_flat, idx_cat)


def _group_bcast_last(x, tlane):
    xm = x * (tlane == T - 1).astype(jnp.float32)
    out = xm
    for t in range(T - 1):
        out = out + jnp.roll(xm, t - (T - 1), axis=1) * (tlane == t).astype(jnp.float32)
    return out


def _main_kernel(cls_ref, resp_ref, bb_ref, tbl_ref, ixyl_ref, tb4_ref,
                 lbl_ref, tibc_ref, tix_ref, tiy_ref, tib_ref, idx_s, out_ref,
                 cls_scr, off_scr, sem):
    copies = []
    for b in range(B):
        for t in range(T):
            bt = b * T + t
            ix = idx_s[bt]
            iy = idx_s[BT + bt]
            copies.append(pltpu.async_copy(
                cls_ref.at[iy, ix, b], cls_scr.at[bt], sem))

    xiota = lax.broadcasted_iota(jnp.int32, (1, W), 1)
    for b in range(B):
        for t in range(T):
            bt = b * T + t
            ix = idx_s[bt]
            iy = idx_s[BT + bt]
            ib = idx_s[2 * BT + bt]
            blk = bb_ref[b, pl.ds(ib * 4, 4), pl.ds(iy, 1), :]
            vals = jnp.sum(blk[:, 0, :] * (xiota == ix).astype(jnp.float32),
                           axis=1, keepdims=True)
            off_scr[:, bt:bt + 1] = vals

    lanev = lax.broadcasted_iota(jnp.int32, (1, BT), 1)
    tlane = lanev % T
    tixf = tix_ref[...].astype(jnp.float32)
    tiyf = tiy_ref[...].astype(jnp.float32)
    tib = tib_ref[...]
    p = tib * S + tiy_ref[...] * W + tix_ref[...]

    dup_later = jnp.zeros((1, BT), jnp.float32)
    dup_earlier = jnp.zeros((1, BT), jnp.float32)
    for d in range(1, T):
        eq_back = (p == jnp.roll(p, d, axis=1)) & (tlane >= d)
        dup_earlier += eq_back.astype(jnp.float32)
        eq_fwd = (p == jnp.roll(p, -d, axis=1)) & (tlane <= T - 1 - d)
        dup_later += eq_fwd.astype(jnp.float32)
    valid_last = (dup_later < 0.5).astype(jnp.float32)
    valid_first = (dup_earlier < 0.5).astype(jnp.float32)

    off = off_scr[...]
    ox, oy = off[0:1, :], off[1:2, :]
    ow, oh = off[2:3, :], off[3:4, :]
    tb4 = tb4_ref[...]
    tbx, tby, tbw, tbh = tb4[0:1], tb4[1:2], tb4[2:3], tb4[3:4]

    px1 = _sigmoid(ox) + tixf - ow * 0.5
    py1 = _sigmoid(oy) + tiyf - oh * 0.5
    gx1 = tbx + tixf - tbw * 0.5
    gy1 = tby + tiyf - tbh * 0.5
    dx = jnp.maximum(jnp.minimum(px1 + ow, gx1 + tbw) - jnp.maximum(px1, gx1), 0.0)
    dy = jnp.maximum(jnp.minimum(py1 + oh, gy1 + tbh) - jnp.maximum(py1, gy1), 0.0)
    inter = dx * dy
    iou_t = inter / (ow * oh + tbw * tbh - inter)

    loss_xy = jnp.sum(_bce(ox, tbx) + _bce(oy, tby))
    loss_wh = jnp.sum((ow - tbw) ** 2 + (oh - tbh) ** 2)

    lgx1 = _group_bcast_last(gx1, tlane)
    lgy1 = _group_bcast_last(gy1, tlane)
    ltbw = _group_bcast_last(tbw, tlane)
    ltbh = _group_bcast_last(tbh, tlane)
    ldx = jnp.maximum(jnp.minimum(px1 + ow, lgx1 + ltbw) - jnp.maximum(px1, lgx1), 0.0)
    ldy = jnp.maximum(jnp.minimum(py1 + oh, lgy1 + ltbh) - jnp.maximum(py1, lgy1), 0.0)
    linter = ldx * ldy
    iou_last_t = linter / (ow * oh + ltbw * ltbh - linter)

    resp_f = resp_ref[...]
    bb4m = bb_ref[...].reshape(B, A, 4, H, W)
    mox, moy = bb4m[:, :, 0], bb4m[:, :, 1]
    mow, moh = bb4m[:, :, 2], bb4m[:, :, 3]
    Xc = lax.broadcasted_iota(jnp.int32, (1, 1, 1, W), 3).astype(jnp.float32)
    Yc = lax.broadcasted_iota(jnp.int32, (1, 1, H, 1), 2).astype(jnp.float32)
    tbl = tbl_ref[...]
    ixyl = ixyl_ref[...].astype(jnp.float32)
    tlx = ixyl[:, 0:1]
    tly = ixyl[:, 1:2]
    Gx1 = (tbl[:, 0:1] + tlx - tbl[:, 2:3] * 0.5)[:, :, None, None]
    Gy1 = (tbl[:, 1:2] + tly - tbl[:, 3:4] * 0.5)[:, :, None, None]
    GW = tbl[:, 2:3][:, :, None, None]
    GH = tbl[:, 3:4][:, :, None, None]
    Px1 = _sigmoid(mox) + Xc - mow * 0.5
    Py1 = _sigmoid(moy) + Yc - moh * 0.5
    DX = jnp.maximum(jnp.minimum(Px1 + mow, Gx1 + GW) - jnp.maximum(Px1, Gx1), 0.0)
    DY = jnp.maximum(jnp.minimum(Py1 + moh, Gy1 + GH) - jnp.maximum(Py1, Gy1), 0.0)
    INTER = DX * DY
    negm = INTER / (mow * moh + GW * GH - INTER) < 0.6
    neg_raw = jnp.sum(jnp.where(negm, _softplus(resp_f), 0.0))

    for cp in copies:
        cp.wait()
    glog100 = cls_scr[...]
    base = tibc_ref[...] * CLS
    ciota = lax.broadcasted_iota(jnp.int32, (BT, C), 1)
    in_rng = (ciota >= base) & (ciota < base + CLS)
    mx = jnp.max(jnp.where(in_rng, glog100, -jnp.inf), axis=1, keepdims=True)
    ex = jnp.where(in_rng, jnp.exp(glog100 - mx), 0.0)
    lse = mx + jnp.log(jnp.sum(ex, axis=1, keepdims=True))
    picked = jnp.sum(jnp.where(ciota == base + lbl_ref[...], glog100, 0.0),
                     axis=1, keepdims=True)
    loss_cls = jnp.sum(lse - picked)

    riota = lax.broadcasted_iota(jnp.int32, (8, BT), 0)
    scal = (jnp.where(lanev == 0, loss_xy, 0.0)
            + jnp.where(lanev == 1, loss_wh, 0.0)
            + jnp.where(lanev == 2, loss_cls, 0.0)
            + jnp.where(lanev == 3, neg_raw, 0.0))
    out = (jnp.where(riota == 0, iou_t, 0.0)
           + jnp.where(riota == 1, iou_last_t, 0.0)
           + jnp.where(riota == 2, valid_last, 0.0)
           + jnp.where(riota == 3, valid_first, 0.0)
           + jnp.where(riota == 4, scal, 0.0))
    out_ref[...] = out


def _epilogue_kernel(out1_ref, gresp_ref, out_ref):
    rows = out1_ref[...]
    resp = gresp_ref[...].reshape(1, BT)
    iou_t = rows[0:1, :]
    iou_last_t = rows[1:2, :]
    valid_last = rows[2:3, :]
    valid_first = rows[3:4, :]
    scal = rows[4:5, :]

    loss_pos = jnp.sum(valid_last * _bce(resp, iou_t))
    sub_neg = jnp.sum(valid_first * jnp.where(iou_last_t < 0.6,
                                              _softplus(resp), 0.0))
    lanev = lax.broadcasted_iota(jnp.int32, (1, BT), 1)

    def pick(k):
        return jnp.sum(jnp.where(lanev == k, scal, 0.0))

    loss_xy, loss_wh, loss_cls, neg_raw = pick(0), pick(1), pick(2), pick(3)
    loss_neg = 0.5 * (neg_raw - sub_neg)

    inv_b = 1.0 / B
    i5 = lax.broadcasted_iota(jnp.int32, (5,), 0)
    out = (jnp.where(i5 == 0, loss_pos * inv_b, 0.0)
           + jnp.where(i5 == 1, loss_neg * inv_b, 0.0)
           + jnp.where(i5 == 2, loss_cls * inv_b, 0.0)
           + jnp.where(i5 == 3, loss_xy * inv_b, 0.0)
           + jnp.where(i5 == 4, loss_wh * inv_b * 5.0, 0.0))
    out_ref[...] = out


@jax.jit
def kernel(pred_cls, pred_response, pred_bboxes, tgt_box, tgt_label, tgt_ix,
           tgt_iy, tgt_ibox):
    tix = tgt_ix.astype(jnp.int32)
    tiy = tgt_iy.astype(jnp.int32)
    tib = tgt_ibox.astype(jnp.int32)
    lbl = tgt_label.astype(jnp.int32)

    idx_cat = jnp.concatenate(
        [tix.reshape(BT), tiy.reshape(BT), tib.reshape(BT)])
    g_resp = _sc_gather(pred_response.reshape(B * A * S), idx_cat)

    smem = pl.BlockSpec(memory_space=pltpu.SMEM)
    out1 = pl.pallas_call(
        _main_kernel,
        out_shape=jax.ShapeDtypeStruct((8, BT), jnp.float32),
        in_specs=[pl.BlockSpec(memory_space=pltpu.HBM),
                  pl.BlockSpec((B, A, H, W), lambda: (0, 0, 0, 0)),
                  pl.BlockSpec((B, 4 * A, H, W), lambda: (0, 0, 0, 0)),
                  pl.BlockSpec((B, 4), lambda: (0, 0)),
                  pl.BlockSpec((B, 2), lambda: (0, 0)),
                  pl.BlockSpec((4, BT), lambda: (0, 0)),
                  pl.BlockSpec((BT, 1), lambda: (0, 0)),
                  pl.BlockSpec((BT, 1), lambda: (0, 0)),
                  pl.BlockSpec((1, BT), lambda: (0, 0)),
                  pl.BlockSpec((1, BT), lambda: (0, 0)),
                  pl.BlockSpec((1, BT), lambda: (0, 0)),
                  smem],
        scratch_shapes=[pltpu.VMEM((BT, C), jnp.float32),
                        pltpu.VMEM((4, BT), jnp.float32),
                        pltpu.SemaphoreType.DMA],
    )(jnp.transpose(pred_cls, (2, 3, 0, 1)), pred_response, pred_bboxes,
      tgt_box[:, T - 1, :], jnp.stack([tix[:, T - 1], tiy[:, T - 1]], axis=1),
      tgt_box.reshape(BT, 4).T, lbl.reshape(BT, 1), tib.reshape(BT, 1),
      tix.reshape(1, BT), tiy.reshape(1, BT), tib.reshape(1, BT), idx_cat)

    out = pl.pallas_call(
        _epilogue_kernel,
        out_shape=jax.ShapeDtypeStruct((5,), jnp.float32),
    )(out1, g_resp)
    return out
